# baseline (speedup 1.0000x reference)
import functools
import math

import jax
import jax.numpy as jnp
from jax import lax
from jax.experimental import pallas as pl
from jax.experimental.pallas import tpu as pltpu
from jax.experimental.pallas import tpu_sc as plsc

BOS_IDX = 2
EOS_IDX = 3
AIR_IDX = 4
LINS_IDX = 5
EMB = 48
F = 12
NW = 32
CH = 128
SCALE = math.sqrt(float(EMB))


def _iota16():
    return lax.broadcasted_iota(jnp.int32, (16,), 0)


def _splat(x):
    return jnp.full((16,), x, jnp.int32)


def _sc_body(feats, emb_flat, wt_flat, b_flat, out_flat, tok_out,
             fbuf, obuf, embbuf, basepat, wtbuf, bbuf, tokbuf):
    wid = lax.axis_index("s") * 2 + lax.axis_index("c")
    r0 = wid * CH
    iota = _iota16()

    a = pl.multiple_of(jnp.maximum(12 * r0 - 16, 0), 8)
    pltpu.sync_copy(feats.at[pl.ds(a, 1552)], fbuf)
    pltpu.sync_copy(emb_flat.at[pl.ds(EMB * BOS_IDX, 4 * EMB)], embbuf)
    pltpu.sync_copy(wt_flat.at[pl.ds(0, 576)], wtbuf)
    pltpu.sync_copy(b_flat.at[pl.ds(0, EMB)], bbuf)

    for c in range(3):
        bb = bbuf[pl.ds(16 * c, 16)]
        basepat[pl.ds(16 * c, 16)] = embbuf[pl.ds(96 + 16 * c, 16)] * SCALE + bb
        basepat[pl.ds(48 + 16 * c, 16)] = embbuf[pl.ds(144 + 16 * c, 16)] * SCALE + bb

    delta = 12 * r0 - 12 - a

    def row(i, carry):
        pb = pl.multiple_of(48 * (i & 1), 16)
        ob = pl.multiple_of(48 * i, 16)
        fbase = 12 * i + delta
        fidx = [jnp.maximum(_splat(fbase + f), 0) for f in range(F)]
        fsp = [plsc.load_gather(fbuf, [fidx[f]]) for f in range(F)]
        for c in range(3):
            acc = basepat[pl.ds(pb + 16 * c, 16)]
            for f in range(F):
                acc = acc + fsp[f] * wtbuf[pl.ds(48 * f + 16 * c, 16)]
            obuf[pl.ds(ob + 16 * c, 16)] = acc
        return carry
    lax.fori_loop(0, CH, row, 0)

    for c in range(8):
        tokbuf[pl.ds(16 * c, 16)] = 4 + (iota & 1)

    @pl.when(wid == 0)
    def _():
        for c in range(3):
            obuf[pl.ds(16 * c, 16)] = embbuf[pl.ds(16 * c, 16)] * SCALE
        tokbuf[pl.ds(0, 16)] = jnp.where(iota == 0, BOS_IDX, 4 + (iota & 1))

    @pl.when(wid == NW - 1)
    def _():
        for c in range(3):
            acc = basepat[pl.ds(16 * c, 16)]
            for f in range(F):
                fs = plsc.load_gather(fbuf, [_splat(1540 + f)])
                acc = acc + fs * wtbuf[pl.ds(48 * f + 16 * c, 16)]
            obuf[pl.ds(6144 + 16 * c, 16)] = acc
            obuf[pl.ds(6192 + 16 * c, 16)] = embbuf[pl.ds(48 + 16 * c, 16)] * SCALE
        tokbuf[pl.ds(CH, 16)] = jnp.where(
            iota == 0, AIR_IDX, jnp.where(iota == 1, EOS_IDX, 0))

    @pl.when(wid < NW - 1)
    def _():
        pltpu.sync_copy(obuf.at[pl.ds(0, 6144)],
                        out_flat.at[pl.ds(pl.multiple_of(r0 * EMB, 8), 6144)])
        pltpu.sync_copy(tokbuf.at[pl.ds(0, CH)],
                        tok_out.at[pl.ds(pl.multiple_of(r0, 8), CH)])

    @pl.when(wid == NW - 1)
    def _():
        pltpu.sync_copy(obuf, out_flat.at[pl.ds(pl.multiple_of(r0 * EMB, 8), 6240)])
        pltpu.sync_copy(tokbuf.at[pl.ds(0, CH)],
                        tok_out.at[pl.ds(pl.multiple_of(r0, 8), CH)])
        pltpu.sync_copy(tokbuf.at[pl.ds(CH, 2)], tok_out.at[pl.ds(NW * CH, 2)])


def kernel(features, embedding, fc_w, fc_b):
    Bn, S, Fd = features.shape
    n_out = S + 2
    feats = features[0].reshape(-1)
    emb_flat = embedding.reshape(-1)
    wt_flat = fc_w.T.reshape(-1)
    mesh = plsc.VectorSubcoreMesh(core_axis_name="c", subcore_axis_name="s")
    k = functools.partial(
        pl.kernel,
        out_type=(
            jax.ShapeDtypeStruct((n_out * EMB,), jnp.float32),
            jax.ShapeDtypeStruct((n_out,), jnp.int32),
        ),
        mesh=mesh,
        compiler_params=pltpu.CompilerParams(needs_layout_passes=False),
        scratch_types=[
            pltpu.VMEM((1552,), jnp.float32),
            pltpu.VMEM((6240,), jnp.float32),
            pltpu.VMEM((192,), jnp.float32),
            pltpu.VMEM((96,), jnp.float32),
            pltpu.VMEM((576,), jnp.float32),
            pltpu.VMEM((48,), jnp.float32),
            pltpu.VMEM((144,), jnp.int32),
        ],
    )(_sc_body)
    out_flat, tok = k(feats, emb_flat, wt_flat, fc_b)
    return out_flat.reshape(1, n_out, EMB), tok.reshape(1, n_out)


# speedup vs baseline: 6.9391x; 6.9391x over previous
import math

import jax
import jax.numpy as jnp
from jax.experimental import pallas as pl

PAD_IDX = 1
BOS_IDX = 2
EOS_IDX = 3
AIR_IDX = 4
LINS_IDX = 5
EMB = 48
BLK = 512


def _tok_embed_kernel(feat_ref, emb_ref, w_ref, b_ref, out_ref, *, n_out):
    scale = math.sqrt(float(EMB))
    blk = out_ref.shape[0]
    pid = pl.program_id(0)
    feat = feat_ref[0]
    w = w_ref[...]
    b = b_ref[0]
    proj = jax.lax.dot_general(
        feat, w, (((1,), (1,)), ((), ())),
        preferred_element_type=jnp.float32) + b[None, :]
    row_bos = emb_ref[BOS_IDX, :] * scale - b
    row_eos = emb_ref[EOS_IDX, :] * scale - b
    row_air = emb_ref[AIR_IDX, :] * scale
    row_lins = emb_ref[LINS_IDX, :] * scale
    i = pid * blk + jax.lax.broadcasted_iota(jnp.int32, (blk, 1), 0)
    base = jnp.where(i % 2 == 0, row_air[None, :], row_lins[None, :])
    base = jnp.where(i == 0, row_bos[None, :], base)
    base = jnp.where(i == n_out - 1, row_eos[None, :], base)
    out_ref[...] = base + proj


def kernel(features, embedding, fc_w, fc_b):
    import functools
    Bn, S, F = features.shape
    n_out = S + 2
    nblk = (n_out + BLK - 1) // BLK
    feat0 = jnp.pad(features[:1], ((0, 0), (1, 1), (0, 0)))
    emb8 = embedding[0:8]
    out = pl.pallas_call(
        functools.partial(_tok_embed_kernel, n_out=n_out),
        out_shape=jax.ShapeDtypeStruct((n_out, EMB), jnp.float32),
        grid=(nblk,),
        in_specs=[
            pl.BlockSpec((1, BLK, F), lambda i: (0, i, 0)),
            pl.BlockSpec((8, EMB), lambda i: (0, 0)),
            pl.BlockSpec((EMB, F), lambda i: (0, 0)),
            pl.BlockSpec((1, EMB), lambda i: (0, 0)),
        ],
        out_specs=pl.BlockSpec((BLK, EMB), lambda i: (i, 0)),
    )(feat0, emb8, fc_w, fc_b.reshape(1, EMB))
    embeddings = out[None]
    pattern = jnp.where(jnp.arange(S) % 2 == 1, AIR_IDX, LINS_IDX).astype(jnp.int32)
    tokens = jnp.concatenate([
        jnp.array([BOS_IDX], dtype=jnp.int32),
        pattern,
        jnp.array([EOS_IDX], dtype=jnp.int32),
    ])[None, :]
    return embeddings, tokens


# speedup vs baseline: 8.7001x; 1.2538x over previous
import math

import jax
import jax.numpy as jnp
from jax.experimental import pallas as pl

PAD_IDX = 1
BOS_IDX = 2
EOS_IDX = 3
AIR_IDX = 4
LINS_IDX = 5
EMB = 48


def _tok_embed_kernel(feat_ref, emb_ref, w_ref, b_ref, out_ref):
    n_out = out_ref.shape[0]
    scale = math.sqrt(float(EMB))
    feat = feat_ref[0]
    w = w_ref[...]
    b = b_ref[0]
    proj = jax.lax.dot_general(
        feat, w, (((1,), (1,)), ((), ())),
        preferred_element_type=jnp.float32) + b[None, :]
    row_bos = emb_ref[BOS_IDX, :] * scale - b
    row_eos = emb_ref[EOS_IDX, :] * scale - b
    row_air = emb_ref[AIR_IDX, :] * scale
    row_lins = emb_ref[LINS_IDX, :] * scale
    i = jax.lax.broadcasted_iota(jnp.int32, (n_out, 1), 0)
    base = jnp.where(i % 2 == 0, row_air[None, :], row_lins[None, :])
    base = jnp.where(i == 0, row_bos[None, :], base)
    base = jnp.where(i == n_out - 1, row_eos[None, :], base)
    out_ref[...] = base + proj


def kernel(features, embedding, fc_w, fc_b):
    Bn, S, F = features.shape
    n_out = S + 2
    feat0 = jnp.pad(features[:1], ((0, 0), (1, 1), (0, 0)))
    emb8 = embedding[0:8]
    out = pl.pallas_call(
        _tok_embed_kernel,
        out_shape=jax.ShapeDtypeStruct((n_out, EMB), jnp.float32),
        grid=(1,),
        in_specs=[
            pl.BlockSpec((1, n_out, F), lambda i: (0, 0, 0)),
            pl.BlockSpec((8, EMB), lambda i: (0, 0)),
            pl.BlockSpec((EMB, F), lambda i: (0, 0)),
            pl.BlockSpec((1, EMB), lambda i: (0, 0)),
        ],
        out_specs=pl.BlockSpec((n_out, EMB), lambda i: (0, 0)),
    )(feat0, emb8, fc_w, fc_b.reshape(1, EMB))
    embeddings = out[None]
    pattern = jnp.where(jnp.arange(S) % 2 == 1, AIR_IDX, LINS_IDX).astype(jnp.int32)
    tokens = jnp.concatenate([
        jnp.array([BOS_IDX], dtype=jnp.int32),
        pattern,
        jnp.array([EOS_IDX], dtype=jnp.int32),
    ])[None, :]
    return embeddings, tokens
